# baseline (speedup 1.0000x reference)
import functools

import jax
import jax.numpy as jnp
from jax import lax
from jax.experimental import pallas as pl
from jax.experimental.pallas import tpu as pltpu
from jax.experimental.pallas import tpu_sc as plsc

_B, _L, _D, _V = 1024, 20, 128, 100000
_NC, _NS, _LANES = 2, 16, 16
_NW = _NC * _NS
_BPW = _B // _NW
_IPW = _BPW * _L
_CHR = 4
_CH = _CHR * _L
_NCHUNK = _BPW // _CHR

_mesh = plsc.VectorSubcoreMesh(core_axis_name="c", subcore_axis_name="s")


@functools.partial(
    pl.kernel,
    mesh=_mesh,
    out_type=jax.ShapeDtypeStruct((_B, _D), jnp.float32),
    scratch_types=[
        pltpu.VMEM((_IPW,), jnp.int32),
        pltpu.VMEM((_NCHUNK, _CH, _D), jnp.float32),
        pltpu.VMEM((_BPW, _D), jnp.float32),
        pltpu.SemaphoreType.DMA((_NCHUNK,)),
    ],
)
def _gather_sum(x_hbm, emb_hbm, h_hbm, idx_v, rows_v, hbuf, sems):
    wid = lax.axis_index("s") * _NC + lax.axis_index("c")
    pltpu.sync_copy(x_hbm.at[pl.ds(wid * _IPW, _IPW)], idx_v)
    copies = [
        pltpu.async_copy(
            emb_hbm.at[idx_v.at[pl.ds(c * _CH, _CH)]], rows_v.at[c],
            sems.at[c])
        for c in range(_NCHUNK)
    ]
    for c in range(_NCHUNK):
        copies[c].wait()

        def body(r, _, c=c):
            rr = r * _L
            for d in range(_D // _LANES):
                sl = pl.ds(d * _LANES, _LANES)
                acc = rows_v[c, rr, sl]
                for l in range(1, _L):
                    acc = acc + rows_v[c, rr + l, sl]
                hbuf[c * _CHR + r, sl] = acc
            return _

        lax.fori_loop(0, _CHR, body, None)
    pltpu.sync_copy(hbuf, h_hbm.at[pl.ds(wid * _BPW, _BPW)])


_VB = 5000
_NVB = _V // _VB


def _mm_body(w_ref, h_ref, o_ref):
    o_ref[...] = lax.dot_general(
        w_ref[...], h_ref[...],
        dimension_numbers=(((1,), (1,)), ((), ())),
        preferred_element_type=jnp.float32)


def _project(h, W):
    out_t = pl.pallas_call(
        _mm_body,
        grid=(_NVB,),
        in_specs=[
            pl.BlockSpec((_VB, _D), lambda i: (i, 0)),
            pl.BlockSpec((_B, _D), lambda i: (0, 0)),
        ],
        out_specs=pl.BlockSpec((_VB, _B), lambda i: (i, 0)),
        out_shape=jax.ShapeDtypeStruct((_V, _B), jnp.float32),
    )(W, h)
    return out_t.T


def kernel(x, emb, W):
    x_flat = x.reshape(-1).astype(jnp.int32)
    h = _gather_sum(x_flat, emb)
    return _project(h, W)


# speedup vs baseline: 1.0495x; 1.0495x over previous
import functools

import jax
import jax.numpy as jnp
from jax import lax
from jax.experimental import pallas as pl
from jax.experimental.pallas import tpu as pltpu
from jax.experimental.pallas import tpu_sc as plsc

_B, _L, _D, _V = 1024, 20, 128, 100000
_NC, _NS, _LANES = 2, 16, 16
_NW = _NC * _NS
_BPW = _B // _NW
_IPW = _BPW * _L
_CHR = 4
_CH = _CHR * _L
_NCHUNK = _BPW // _CHR

_mesh = plsc.VectorSubcoreMesh(core_axis_name="c", subcore_axis_name="s")


@functools.partial(
    pl.kernel,
    mesh=_mesh,
    out_type=jax.ShapeDtypeStruct((_B, _D), jnp.float32),
    scratch_types=[
        pltpu.VMEM((_IPW,), jnp.int32),
        pltpu.VMEM((_NCHUNK, _CH, _D), jnp.float32),
        pltpu.VMEM((_BPW, _D), jnp.float32),
        pltpu.SemaphoreType.DMA,
    ],
)
def _gather_sum(x_hbm, emb_hbm, h_hbm, idx_v, rows_v, hbuf, sem):
    wid = lax.axis_index("s") * _NC + lax.axis_index("c")
    pltpu.sync_copy(x_hbm.at[pl.ds(wid * _IPW, _IPW)], idx_v)
    copies = [
        pltpu.async_copy(
            emb_hbm.at[idx_v.at[pl.ds(c * _CH, _CH)]], rows_v.at[c], sem)
        for c in range(_NCHUNK)
    ]
    for cp in copies:
        cp.wait()

    def body(r, _):
        c = r // _CHR
        rr = (r % _CHR) * _L
        for d in range(_D // _LANES):
            sl = pl.ds(d * _LANES, _LANES)
            acc = rows_v[c, rr, sl]
            for l in range(1, _L):
                acc = acc + rows_v[c, rr + l, sl]
            hbuf[r, sl] = acc
        return _

    lax.fori_loop(0, _BPW, body, None)
    pltpu.sync_copy(hbuf, h_hbm.at[pl.ds(wid * _BPW, _BPW)])


_VB = 4000
_NVB = _V // _VB
_K = 3


def _mm_body(w_ref, h_ref, o_hbm, ring, sems):
    i = pl.program_id(0)
    s = lax.rem(i, _K)

    @pl.when(i >= _K)
    def _():
        pltpu.make_async_copy(
            ring.at[s], o_hbm.at[pl.ds((i - _K) * _VB, _VB), :], sems.at[s]
        ).wait()

    ring[s] = lax.dot_general(
        w_ref[...], h_ref[...],
        dimension_numbers=(((1,), (1,)), ((), ())),
        preferred_element_type=jnp.float32)
    pltpu.make_async_copy(
        ring.at[s], o_hbm.at[pl.ds(i * _VB, _VB), :], sems.at[s]
    ).start()

    @pl.when(i == _NVB - 1)
    def _():
        for j in range(_K):
            sj = lax.rem(i - j + _K, _K)
            pltpu.make_async_copy(
                ring.at[sj], o_hbm.at[pl.ds((i - j) * _VB, _VB), :],
                sems.at[sj]
            ).wait()


def _project(h, W):
    out_t = pl.pallas_call(
        _mm_body,
        grid=(_NVB,),
        in_specs=[
            pl.BlockSpec((_VB, _D), lambda i: (i, 0)),
            pl.BlockSpec((_B, _D), lambda i: (0, 0)),
        ],
        out_specs=pl.BlockSpec(memory_space=pl.ANY),
        out_shape=jax.ShapeDtypeStruct((_V, _B), jnp.float32),
        scratch_shapes=[
            pltpu.VMEM((_K, _VB, _B), jnp.float32),
            pltpu.SemaphoreType.DMA((_K,)),
        ],
        compiler_params=pltpu.CompilerParams(
            vmem_limit_bytes=100 * 1024 * 1024),
    )(W, h)
    return out_t.T


def kernel(x, emb, W):
    x_flat = x.reshape(-1).astype(jnp.int32)
    h = _gather_sum(x_flat, emb)
    return _project(h, W)


# speedup vs baseline: 1.0548x; 1.0051x over previous
import functools

import jax
import jax.numpy as jnp
from jax import lax
from jax.experimental import pallas as pl
from jax.experimental.pallas import tpu as pltpu
from jax.experimental.pallas import tpu_sc as plsc

_B, _L, _D, _V = 1024, 20, 128, 100000
_NC, _NS, _LANES = 2, 16, 16
_NW = _NC * _NS
_BPW = _B // _NW
_IPW = _BPW * _L
_CHR = 4
_CH = _CHR * _L
_NCHUNK = _BPW // _CHR

_mesh = plsc.VectorSubcoreMesh(core_axis_name="c", subcore_axis_name="s")


@functools.partial(
    pl.kernel,
    mesh=_mesh,
    out_type=jax.ShapeDtypeStruct((_B, _D), jnp.float32),
    scratch_types=[
        pltpu.VMEM((_IPW,), jnp.int32),
        pltpu.VMEM((_NCHUNK, _CH, _D), jnp.float32),
        pltpu.VMEM((_BPW, _D), jnp.float32),
        pltpu.SemaphoreType.DMA,
    ],
)
def _gather_sum(x_hbm, emb_hbm, h_hbm, idx_v, rows_v, hbuf, sem):
    wid = lax.axis_index("s") * _NC + lax.axis_index("c")
    pltpu.sync_copy(x_hbm.at[pl.ds(wid * _IPW, _IPW)], idx_v)
    copies = [
        pltpu.async_copy(
            emb_hbm.at[idx_v.at[pl.ds(c * _CH, _CH)]], rows_v.at[c], sem)
        for c in range(_NCHUNK)
    ]
    for cp in copies:
        cp.wait()

    def body(r, _):
        c = r // _CHR
        rr = (r % _CHR) * _L
        for d in range(_D // _LANES):
            sl = pl.ds(d * _LANES, _LANES)
            acc = rows_v[c, rr, sl]
            for l in range(1, _L):
                acc = acc + rows_v[c, rr + l, sl]
            hbuf[r, sl] = acc
        return _

    lax.fori_loop(0, _BPW, body, None)
    pltpu.sync_copy(hbuf, h_hbm.at[pl.ds(wid * _BPW, _BPW)])


_VB = 5000
_NVB = _V // _VB


def _mm_body(w_ref, h_ref, o_ref):
    o_ref[...] = lax.dot_general(
        w_ref[...], h_ref[...],
        dimension_numbers=(((1,), (1,)), ((), ())),
        preferred_element_type=jnp.float32)


def _project(h, W):
    out_t = pl.pallas_call(
        _mm_body,
        grid=(_NVB,),
        in_specs=[
            pl.BlockSpec((_VB, _D), lambda i: (i, 0)),
            pl.BlockSpec((_B, _D), lambda i: (0, 0)),
        ],
        out_specs=pl.BlockSpec((_VB, _B), lambda i: (i, 0)),
        out_shape=jax.ShapeDtypeStruct((_V, _B), jnp.float32),
    )(W, h)
    return out_t.T


def kernel(x, emb, W):
    x_flat = x.reshape(-1).astype(jnp.int32)
    h = _gather_sum(x_flat, emb)
    return _project(h, W)


# speedup vs baseline: 1.0636x; 1.0083x over previous
import functools

import jax
import jax.numpy as jnp
from jax import lax
from jax.experimental import pallas as pl
from jax.experimental.pallas import tpu as pltpu
from jax.experimental.pallas import tpu_sc as plsc

_B, _L, _D, _V = 1024, 20, 128, 100000
_NC, _NS, _LANES = 2, 16, 16
_NW = _NC * _NS
_BPW = _B // _NW
_IPW = _BPW * _L
_CH = 128
_NCHUNK = _IPW // _CH

_mesh = plsc.VectorSubcoreMesh(core_axis_name="c", subcore_axis_name="s")


@functools.partial(
    pl.kernel,
    mesh=_mesh,
    out_type=jax.ShapeDtypeStruct((_B, _D), jnp.float32),
    scratch_types=[
        pltpu.VMEM((_IPW,), jnp.int32),
        pltpu.VMEM((_IPW, _D), jnp.float32),
        pltpu.VMEM((_BPW, _D), jnp.float32),
        pltpu.SemaphoreType.DMA,
    ],
)
def _gather_sum(x_hbm, emb_hbm, h_hbm, idx_v, rows_v, hbuf, sem):
    wid = lax.axis_index("s") * _NC + lax.axis_index("c")
    pltpu.sync_copy(x_hbm.at[pl.ds(wid * _IPW, _IPW)], idx_v)
    copies = [
        pltpu.async_copy(
            emb_hbm.at[idx_v.at[pl.ds(c * _CH, _CH)]],
            rows_v.at[pl.ds(c * _CH, _CH)], sem)
        for c in range(_NCHUNK)
    ]
    for cp in copies:
        cp.wait()

    def body(r, _):
        rr = r * _L
        for d in range(_D // _LANES):
            sl = pl.ds(d * _LANES, _LANES)
            acc = rows_v[rr, sl]
            for l in range(1, _L):
                acc = acc + rows_v[rr + l, sl]
            hbuf[r, sl] = acc
        return _

    lax.fori_loop(0, _BPW, body, None)
    pltpu.sync_copy(hbuf, h_hbm.at[pl.ds(wid * _BPW, _BPW)])


_VB = 5000
_NVB = _V // _VB


def _mm_body(w_ref, h_ref, o_ref):
    o_ref[...] = lax.dot_general(
        w_ref[...], h_ref[...],
        dimension_numbers=(((1,), (1,)), ((), ())),
        preferred_element_type=jnp.float32)


def _project(h, W):
    out_t = pl.pallas_call(
        _mm_body,
        grid=(_NVB,),
        in_specs=[
            pl.BlockSpec((_VB, _D), lambda i: (i, 0)),
            pl.BlockSpec((_B, _D), lambda i: (0, 0)),
        ],
        out_specs=pl.BlockSpec((_VB, _B), lambda i: (i, 0)),
        out_shape=jax.ShapeDtypeStruct((_V, _B), jnp.float32),
    )(W, h)
    return out_t.T


def kernel(x, emb, W):
    x_flat = x.reshape(-1).astype(jnp.int32)
    h = _gather_sum(x_flat, emb)
    return _project(h, W)
